# baseline (speedup 1.0000x reference)
import functools

import jax
import jax.numpy as jnp
from jax import lax
from jax.experimental import pallas as pl
from jax.experimental.pallas import tpu as pltpu
from jax.experimental.pallas import tpu_sc as plsc

D_MODEL = 768
LANES = 16
NUM_CORES = 2
NUM_SUBCORES = 16
NUM_WORKERS = NUM_CORES * NUM_SUBCORES
CHUNK_ROWS = 16
NBUF = 4


def _sc_body(
    x_hbm, pos_hbm, scale_hbm, out_hbm,
    pbuf, xin0, xin1, xin2, xin3, xout0, xout1, xout2, xout3, sbuf,
    insem0, insem1, insem2, insem3, outsem0, outsem1, outsem2, outsem3,
):
    wid = lax.axis_index("s") * NUM_CORES + lax.axis_index("c")
    B, S, _ = x_hbm.shape
    pos_rows_per_worker = S // NUM_WORKERS
    prow0 = wid * pos_rows_per_worker
    num_pc = pos_rows_per_worker // CHUNK_ROWS
    num_pairs = num_pc * B

    xins = (xin0, xin1, xin2, xin3)
    xouts = (xout0, xout1, xout2, xout3)
    insems = (insem0, insem1, insem2, insem3)
    outsems = (outsem0, outsem1, outsem2, outsem3)

    def start_in(t, j):
        b = t % B
        row = prow0 + (t // B) * CHUNK_ROWS
        pltpu.make_async_copy(
            x_hbm.at[b, pl.ds(row, CHUNK_ROWS), :], xins[j], insems[j]
        ).start()

    pltpu.sync_copy(scale_hbm, sbuf)
    sv = sbuf[...]

    for j in range(NBUF):
        start_in(j, j)

    def group_body(g, carry):
        pltpu.sync_copy(
            pos_hbm.at[pl.ds(prow0 + g * CHUNK_ROWS, CHUNK_ROWS), :], pbuf
        )
        for u in range(B):
            t = B * g + u
            j = u

            pltpu.make_async_copy(
                x_hbm.at[0, pl.ds(0, CHUNK_ROWS), :], xins[j], insems[j]
            ).wait()

            @pl.when(g >= 1)
            def _():
                pltpu.make_async_copy(
                    xouts[j], out_hbm.at[0, pl.ds(0, CHUNK_ROWS), :], outsems[j]
                ).wait()

            xin = xins[j]
            xout = xouts[j]

            @plsc.parallel_loop(0, CHUNK_ROWS, 1, unroll=2)
            def _(r):
                for v in range(D_MODEL // LANES):
                    sl = pl.ds(v * LANES, LANES)
                    xout[r, sl] = xin[r, sl] + pbuf[r, sl] * sv

            b = t % B
            row = prow0 + (t // B) * CHUNK_ROWS
            pltpu.make_async_copy(
                xout, out_hbm.at[b, pl.ds(row, CHUNK_ROWS), :], outsems[j]
            ).start()

            @pl.when(g + 1 < num_pc)
            def _():
                start_in(t + NBUF, j)
        return carry

    lax.fori_loop(0, num_pc, group_body, 0)

    for j in range(NBUF):
        pltpu.make_async_copy(
            xouts[j], out_hbm.at[0, pl.ds(0, CHUNK_ROWS), :], outsems[j]
        ).wait()


def kernel(x, pos_embed, scale):
    B, S, D = x.shape
    mesh = plsc.VectorSubcoreMesh(core_axis_name="c", subcore_axis_name="s")

    sc_call = functools.partial(
        pl.kernel,
        mesh=mesh,
        out_type=jax.ShapeDtypeStruct((B, S, D), jnp.float32),
        compiler_params=pltpu.CompilerParams(use_tc_tiling_on_sc=True),
        scratch_types=(
            [pltpu.VMEM((CHUNK_ROWS, D_MODEL), jnp.float32)] * 9
            + [pltpu.VMEM((LANES,), jnp.float32)]
            + [pltpu.SemaphoreType.DMA] * 8
        ),
    )(_sc_body)

    return sc_call(x, pos_embed[:S], jnp.broadcast_to(scale, (LANES,)))
